# baseline (speedup 1.0000x reference)
import functools

import jax
import jax.numpy as jnp
from jax import lax
from jax.experimental import pallas as pl
from jax.experimental.pallas import tpu as pltpu
from jax.experimental.pallas import tpu_sc as plsc

H = 4096
W = 4096
C = 3
PLANE = H * W
N_SAMPLES = 1_000_000

NC = 2
NS = 16
NW = NC * NS
LANES = 16

B = 512
CHUNKS = 63
NPW = B * CHUNKS
NPAD = NPW * NW

_SET = lambda: [
    pltpu.VMEM((B,), jnp.float32),
    pltpu.VMEM((B,), jnp.float32),
    [pltpu.VMEM((B,), jnp.int32) for _ in range(12)],
    [pltpu.VMEM((B,), jnp.float32) for _ in range(4)],
    [pltpu.VMEM((B,), jnp.float32) for _ in range(12)],
    [pltpu.VMEM((B,), jnp.float32) for _ in range(3)],
]


@functools.partial(
    pl.kernel,
    mesh=plsc.VectorSubcoreMesh(core_axis_name="c", subcore_axis_name="s"),
    compiler_params=pltpu.CompilerParams(
        needs_layout_passes=False, use_tc_tiling_on_sc=False),
    out_type=jax.ShapeDtypeStruct((C * NPAD,), jnp.float32),
    scratch_types=[
        _SET(), _SET(), _SET(),
        pltpu.SemaphoreType.DMA,
        pltpu.SemaphoreType.DMA,
        pltpu.SemaphoreType.DMA,
        pltpu.SemaphoreType.DMA,
        pltpu.SemaphoreType.DMA,
    ],
)
def _bilerp(xsx_hbm, xsy_hbm, data_hbm, out_hbm,
            set0, set1, set2, semg0, semg1, semg2, semx, semo):
    wid = lax.axis_index("s") * NC + lax.axis_index("c")
    wbase = wid * NPW
    sets = (set0, set1, set2)
    semg = (semg0, semg1, semg2)

    def fire_xs(k, s):
        xx_v, yy_v = sets[s][0], sets[s][1]
        base = wbase + k * B
        pltpu.sync_copy(xsx_hbm.at[pl.ds(base, B)], xx_v)
        pltpu.sync_copy(xsy_hbm.at[pl.ds(base, B)], yy_v)

    def wait_xs(s):
        pass

    def ph1(s):
        xx_v, yy_v, idx_v, w_v, _, _ = sets[s]

        def body(j, _):
            sl = pl.ds(j * LANES, LANES)
            sx = xx_v[sl] * jnp.float32(W)
            sy = yy_v[sl] * jnp.float32(H)
            ix = sx.astype(jnp.int32)
            iy = sy.astype(jnp.int32)
            fx = sx - ix.astype(jnp.float32)
            fy = sy - iy.astype(jnp.float32)
            x0 = jnp.clip(ix, 0, W - 1)
            y0 = jnp.clip(iy, 0, H - 1)
            x1 = jnp.minimum(x0 + 1, W - 1)
            y1 = jnp.minimum(y0 + 1, H - 1)
            yb0 = y0 << 12
            yb1 = y1 << 12
            e = [yb0 + x0, yb0 + x1, yb1 + x0, yb1 + x1]
            for kk in range(4):
                for cc in range(C):
                    idx_v[kk * C + cc][sl] = e[kk] + (cc * PLANE)
            gx = 1.0 - fx
            gy = 1.0 - fy
            w_v[0][sl] = gx * gy
            w_v[1][sl] = fx * gy
            w_v[2][sl] = gx * fy
            w_v[3][sl] = fx * fy
            return 0

        lax.fori_loop(0, B // LANES, body, 0, unroll=2)

    def fire_g(s):
        idx_v, g_v = sets[s][2], sets[s][4]
        for q in range(12):
            pltpu.async_copy(data_hbm.at[idx_v[q]], g_v[q], semg[s])

    def wait_g(s):
        idx_v, g_v = sets[s][2], sets[s][4]
        for q in range(12):
            pltpu.make_async_copy(data_hbm.at[idx_v[q]], g_v[q], semg[s]).wait()

    def ph3(k, s):
        w_v, g_v, o_v = sets[s][3], sets[s][4], sets[s][5]
        base = wbase + k * B

        def body(j, _):
            sl = pl.ds(j * LANES, LANES)
            ws = [w_v[kk][sl] for kk in range(4)]
            for cc in range(C):
                acc = g_v[cc][sl] * ws[0]
                acc = acc + g_v[C + cc][sl] * ws[1]
                acc = acc + g_v[2 * C + cc][sl] * ws[2]
                acc = acc + g_v[3 * C + cc][sl] * ws[3]
                o_v[cc][sl] = acc
            return 0

        lax.fori_loop(0, B // LANES, body, 0, unroll=2)
        for cc in range(C):
            pltpu.sync_copy(o_v[cc], out_hbm.at[pl.ds(cc * NPAD + base, B)])

    def wait_o(k, s):
        pass

    fire_xs(0, 0)
    ph1(0)
    fire_g(0)
    fire_xs(1, 1)
    ph1(1)
    fire_g(1)

    def triple(i, _):
        for j in range(3):
            k = 3 * i + j
            s = j
            t = (j + 2) % 3
            wait_g(s)

            @pl.when(k >= 3)
            def _():
                wait_o(k - 3, s)

            ph3(k, s)

            @pl.when(k + 2 < CHUNKS)
            def _():
                fire_xs(k + 2, t)
                ph1(t)
                fire_g(t)
        return 0

    lax.fori_loop(0, CHUNKS // 3, triple, 0)
    for k in (CHUNKS - 3, CHUNKS - 2, CHUNKS - 1):
        wait_o(k, k % 3)


def kernel(xs, data):
    npad = NPAD - N_SAMPLES
    spread = (jnp.arange(npad, dtype=jnp.float32) % 4093.0) / 4096.0
    xs_x = jnp.concatenate([xs[:, 0], spread])
    xs_y = jnp.concatenate([xs[:, 1], spread])
    table = data.transpose(2, 0, 1).reshape(49152, 8, 128).reshape(C * PLANE)
    out_planar = _bilerp(xs_x, xs_y, table)
    return out_planar.reshape(C, NPAD)[:, :N_SAMPLES].T


# speedup vs baseline: 1.0094x; 1.0094x over previous
import functools

import jax
import jax.numpy as jnp
from jax import lax
from jax.experimental import pallas as pl
from jax.experimental.pallas import tpu as pltpu
from jax.experimental.pallas import tpu_sc as plsc

H = 4096
W = 4096
C = 3
PLANE = H * W
N_SAMPLES = 1_000_000

NC = 2
NS = 16
NW = NC * NS
LANES = 16

B = 512
CHUNKS = 63
NPW = B * CHUNKS
NPAD = NPW * NW

_SET = lambda: [
    [pltpu.VMEM((B,), jnp.int32) for _ in range(12)],
    [pltpu.VMEM((B,), jnp.float32) for _ in range(4)],
    [pltpu.VMEM((B,), jnp.float32) for _ in range(12)],
    [pltpu.VMEM((B,), jnp.float32) for _ in range(3)],
]


@functools.partial(
    pl.kernel,
    mesh=plsc.VectorSubcoreMesh(core_axis_name="c", subcore_axis_name="s"),
    compiler_params=pltpu.CompilerParams(
        needs_layout_passes=False, use_tc_tiling_on_sc=False),
    out_type=jax.ShapeDtypeStruct((C * NPAD,), jnp.float32),
    scratch_types=[
        pltpu.VMEM((NPW,), jnp.float32),
        pltpu.VMEM((NPW,), jnp.float32),
        _SET(), _SET(), _SET(),
        [pltpu.SemaphoreType.DMA for _ in range(3)],
        [pltpu.SemaphoreType.DMA for _ in range(3)],
    ],
)
def _bilerp(xsx_hbm, xsy_hbm, data_hbm, out_hbm,
            xx_all, yy_all, set0, set1, set2, semg, semo):
    wid = lax.axis_index("s") * NC + lax.axis_index("c")
    wbase = wid * NPW
    sets = (set0, set1, set2)

    def ph1(k, s):
        idx_v, w_v = sets[s][0], sets[s][1]
        koff = k * B

        def body(j, _):
            sl = pl.ds(j * LANES, LANES)
            xsl = pl.ds(koff + j * LANES, LANES)
            sx = xx_all[xsl] * jnp.float32(W)
            sy = yy_all[xsl] * jnp.float32(H)
            ix = sx.astype(jnp.int32)
            iy = sy.astype(jnp.int32)
            fx = sx - ix.astype(jnp.float32)
            fy = sy - iy.astype(jnp.float32)
            x0 = jnp.clip(ix, 0, W - 1)
            y0 = jnp.clip(iy, 0, H - 1)
            x1 = jnp.minimum(x0 + 1, W - 1)
            y1 = jnp.minimum(y0 + 1, H - 1)
            yb0 = y0 << 12
            yb1 = y1 << 12
            e = [yb0 + x0, yb0 + x1, yb1 + x0, yb1 + x1]
            for kk in range(4):
                for cc in range(C):
                    idx_v[kk * C + cc][sl] = e[kk] + (cc * PLANE)
            gx = 1.0 - fx
            gy = 1.0 - fy
            w_v[0][sl] = gx * gy
            w_v[1][sl] = fx * gy
            w_v[2][sl] = gx * fy
            w_v[3][sl] = fx * fy
            return 0

        lax.fori_loop(0, B // LANES, body, 0, unroll=2)

    def fire_g(s):
        idx_v, g_v = sets[s][0], sets[s][2]
        for q in range(12):
            pltpu.async_copy(data_hbm.at[idx_v[q]], g_v[q], semg[s])

    def wait_g(s):
        idx_v, g_v = sets[s][0], sets[s][2]
        for q in range(12):
            pltpu.make_async_copy(data_hbm.at[idx_v[q]], g_v[q], semg[s]).wait()

    def ph3(k, s):
        w_v, g_v, o_v = sets[s][1], sets[s][2], sets[s][3]
        base = wbase + k * B

        def body(j, _):
            sl = pl.ds(j * LANES, LANES)
            ws = [w_v[kk][sl] for kk in range(4)]
            for cc in range(C):
                acc = g_v[cc][sl] * ws[0]
                acc = acc + g_v[C + cc][sl] * ws[1]
                acc = acc + g_v[2 * C + cc][sl] * ws[2]
                acc = acc + g_v[3 * C + cc][sl] * ws[3]
                o_v[cc][sl] = acc
            return 0

        lax.fori_loop(0, B // LANES, body, 0, unroll=2)
        for cc in range(C):
            pltpu.async_copy(o_v[cc], out_hbm.at[pl.ds(cc * NPAD + base, B)],
                             semo[s])

    def wait_o(k, s):
        o_v = sets[s][3]
        base = wbase + k * B
        for cc in range(C):
            pltpu.make_async_copy(
                o_v[cc], out_hbm.at[pl.ds(cc * NPAD + base, B)], semo[s]).wait()

    pltpu.sync_copy(xsx_hbm.at[pl.ds(wbase, NPW)], xx_all)
    pltpu.sync_copy(xsy_hbm.at[pl.ds(wbase, NPW)], yy_all)
    ph1(0, 0)
    fire_g(0)
    ph1(1, 1)
    fire_g(1)

    def triple(i, _):
        for j in range(3):
            k = 3 * i + j
            s = j
            t = (j + 2) % 3
            wait_g(s)

            @pl.when(k >= 3)
            def _():
                wait_o(k - 3, s)

            ph3(k, s)

            @pl.when(k + 2 < CHUNKS)
            def _():
                ph1(k + 2, t)
                fire_g(t)
        return 0

    lax.fori_loop(0, CHUNKS // 3, triple, 0)
    for k in (CHUNKS - 3, CHUNKS - 2, CHUNKS - 1):
        wait_o(k, k % 3)


def kernel(xs, data):
    npad = NPAD - N_SAMPLES
    spread = (jnp.arange(npad, dtype=jnp.float32) % 4093.0) / 4096.0
    xs_x = jnp.concatenate([xs[:, 0], spread])
    xs_y = jnp.concatenate([xs[:, 1], spread])
    table = data.transpose(2, 0, 1).reshape(49152, 8, 128).reshape(C * PLANE)
    out_planar = _bilerp(xs_x, xs_y, table)
    return out_planar.reshape(C, NPAD)[:, :N_SAMPLES].T
